# initial kernel scaffold (unmeasured)
import jax
import jax.numpy as jnp
from jax import lax
from jax.experimental import pallas as pl
from jax.experimental.pallas import tpu as pltpu

N_DEV = 4


def _ar_body(p_ref, y_ref, comm, send_sems, recv_sems, pass_sem):
    del p_ref
    m, n = y_ref.shape
    chunk = m // N_DEV
    half = n // 2

    my = lax.axis_index("i")
    left = (my + N_DEV - 1) % N_DEV
    right = (my + 1) % N_DEV

    barrier = pltpu.get_barrier_semaphore()
    for nbr in (left, right):
        pl.semaphore_signal(
            barrier, inc=1, device_id=(nbr,),
            device_id_type=pl.DeviceIdType.MESH,
        )
    pl.semaphore_wait(barrier, 2)

    for p in range(2):
        col = pl.ds(p * half, half)

        for s in range(N_DEV - 1):
            c_send = (my + N_DEV - s) % N_DEV
            c_recv = (my + N_DEV - s - 1) % N_DEV
            rdma = pltpu.make_async_remote_copy(
                src_ref=y_ref.at[pl.ds(c_send * chunk, chunk), col],
                dst_ref=comm.at[s],
                send_sem=send_sems.at[s],
                recv_sem=recv_sems.at[s],
                device_id=(right,),
                device_id_type=pl.DeviceIdType.MESH,
            )
            rdma.start()
            rdma.wait()
            rows = pl.ds(c_recv * chunk, chunk)
            acc = y_ref[rows, col].astype(jnp.float32) + comm[s].astype(
                jnp.float32
            )
            y_ref[rows, col] = acc.astype(jnp.bfloat16)

        for s in range(N_DEV - 1):
            c_send = (my + N_DEV + 1 - s) % N_DEV
            rows = pl.ds(c_send * chunk, chunk)
            rdma = pltpu.make_async_remote_copy(
                src_ref=y_ref.at[rows, col],
                dst_ref=y_ref.at[rows, col],
                send_sem=send_sems.at[3 + s],
                recv_sem=recv_sems.at[3 + s],
                device_id=(right,),
                device_id_type=pl.DeviceIdType.MESH,
            )
            rdma.start()
            rdma.wait()

        if p == 0:
            for nbr in (left, right):
                pl.semaphore_signal(
                    pass_sem, inc=1, device_id=(nbr,),
                    device_id_type=pl.DeviceIdType.MESH,
                )
            pl.semaphore_wait(pass_sem, 2)


def kernel(x, w_mat):
    m = x.shape[0]
    n = w_mat.shape[1]
    part = jnp.dot(
        x, w_mat, preferred_element_type=jnp.float32
    ).astype(jnp.bfloat16)

    y = pl.pallas_call(
        _ar_body,
        out_shape=jax.ShapeDtypeStruct((m, n), jnp.bfloat16),
        in_specs=[pl.BlockSpec(memory_space=pltpu.VMEM)],
        out_specs=pl.BlockSpec(memory_space=pltpu.VMEM),
        scratch_shapes=[
            pltpu.VMEM((N_DEV - 1, m // N_DEV, n // 2), jnp.bfloat16),
            pltpu.SemaphoreType.DMA((6,)),
            pltpu.SemaphoreType.DMA((6,)),
            pltpu.SemaphoreType.REGULAR,
        ],
        input_output_aliases={0: 0},
        compiler_params=pltpu.CompilerParams(collective_id=0),
    )(part)

    yf = y.astype(jnp.float32)
    scale = jnp.max(jnp.abs(yf)) / 448.0
    q = jnp.clip(yf / scale, -448.0, 448.0).astype(jnp.float8_e4m3fn)
    return q.astype(jnp.float32) * scale


# baseline (device time: 2568284 ns/iter reference)
import jax
import jax.numpy as jnp
from jax import lax
from jax.experimental import pallas as pl
from jax.experimental.pallas import tpu as pltpu

N_DEV = 4
N_PASS = 8


def _ar_body(
    p_ref, y_ref, send_buf, own_buf, rs_bufs, ag_bufs,
    send_sems, recv_sems, loc_a, loc_b, pass_sem,
):
    m, n = p_ref.shape
    chunk = m // N_DEV
    width = n // N_PASS

    my = lax.axis_index("i")
    left = (my + N_DEV - 1) % N_DEV
    right = (my + 1) % N_DEV

    barrier = pltpu.get_barrier_semaphore()
    for nbr in (left, right):
        pl.semaphore_signal(
            barrier, inc=1, device_id=(nbr,),
            device_id_type=pl.DeviceIdType.MESH,
        )
    pl.semaphore_wait(barrier, 2)

    def rows(c):
        return pl.ds(c * chunk, chunk)

    for p in range(N_PASS):
        col = pl.ds(p * width, width)

        cp = pltpu.make_async_copy(p_ref.at[rows(my), col], send_buf, loc_a)
        cp.start()
        cp.wait()

        for s in range(N_DEV - 1):
            c_recv = (my + N_DEV - s - 1) % N_DEV
            rdma = pltpu.make_async_remote_copy(
                src_ref=send_buf,
                dst_ref=rs_bufs.at[s],
                send_sem=send_sems.at[s],
                recv_sem=recv_sems.at[s],
                device_id=(right,),
                device_id_type=pl.DeviceIdType.MESH,
            )
            rdma.start()
            cp = pltpu.make_async_copy(
                p_ref.at[rows(c_recv), col], own_buf, loc_b
            )
            cp.start()
            rdma.wait()
            cp.wait()
            send_buf[...] = rs_bufs[s] + own_buf[...]

        cp = pltpu.make_async_copy(
            send_buf, y_ref.at[rows((my + 1) % N_DEV), col], loc_a
        )
        cp.start()
        cp.wait()

        src = send_buf
        for s in range(N_DEV - 1):
            rdma = pltpu.make_async_remote_copy(
                src_ref=src,
                dst_ref=ag_bufs.at[s],
                send_sem=send_sems.at[3 + s],
                recv_sem=recv_sems.at[3 + s],
                device_id=(right,),
                device_id_type=pl.DeviceIdType.MESH,
            )
            rdma.start()
            rdma.wait()
            cp = pltpu.make_async_copy(
                ag_bufs.at[s], y_ref.at[rows((my + N_DEV - s) % N_DEV), col],
                loc_a,
            )
            cp.start()
            cp.wait()
            src = ag_bufs.at[s]

        if p < N_PASS - 1:
            for nbr in (left, right):
                pl.semaphore_signal(
                    pass_sem, inc=1, device_id=(nbr,),
                    device_id_type=pl.DeviceIdType.MESH,
                )
            pl.semaphore_wait(pass_sem, 2)


def _all_reduce(part):
    m, n = part.shape
    chunk = m // N_DEV
    width = n // N_PASS
    return pl.pallas_call(
        _ar_body,
        out_shape=jax.ShapeDtypeStruct((m, n), jnp.float32),
        in_specs=[pl.BlockSpec(memory_space=pl.ANY)],
        out_specs=pl.BlockSpec(memory_space=pl.ANY),
        scratch_shapes=[
            pltpu.VMEM((chunk, width), jnp.float32),
            pltpu.VMEM((chunk, width), jnp.float32),
            pltpu.VMEM((N_DEV - 1, chunk, width), jnp.float32),
            pltpu.VMEM((N_DEV - 1, chunk, width), jnp.float32),
            pltpu.SemaphoreType.DMA((6,)),
            pltpu.SemaphoreType.DMA((6,)),
            pltpu.SemaphoreType.DMA,
            pltpu.SemaphoreType.DMA,
            pltpu.SemaphoreType.REGULAR,
        ],
        compiler_params=pltpu.CompilerParams(collective_id=0),
    )(part)


def kernel(x, w_mat):
    part = jnp.dot(x, w_mat, preferred_element_type=jnp.float32)
    y = _all_reduce(part)

    yf = y
    scale = jnp.max(jnp.abs(yf)) / 448.0
    q = jnp.clip(yf / scale, -448.0, 448.0).astype(jnp.float8_e4m3fn)
    q = lax.optimization_barrier(q)
    return q.astype(jnp.float32) * scale


# device time: 646367 ns/iter; 3.9734x vs baseline; 3.9734x over previous
import jax
import jax.numpy as jnp
from jax import lax
from jax.experimental import pallas as pl
from jax.experimental.pallas import tpu as pltpu

N_DEV = 4
N_PASS = 4


def _body(
    p_ref, q_ref, amax_ref, ytmp_ref,
    send_cw, own_cw, rs_cw,
    send_ccw, own_ccw, rs_ccw,
    qbuf, rmax, amax_buf,
    s_cw, r_cw, s_ccw, r_ccw,
    am_s, am_r, ag_s_cw, ag_r_cw, ag_s_ccw, ag_r_ccw,
    loc_a, loc_b, loc_c, loc_d, pass_sem,
):
    m, n = p_ref.shape
    chunk = m // N_DEV
    halfn = n // 2
    w = halfn // N_PASS

    my = lax.axis_index("i")
    left = (my + N_DEV - 1) % N_DEV
    right = (my + 1) % N_DEV

    barrier = pltpu.get_barrier_semaphore()
    for nbr in (left, right):
        pl.semaphore_signal(
            barrier, inc=1, device_id=(nbr,),
            device_id_type=pl.DeviceIdType.MESH,
        )
    pl.semaphore_wait(barrier, 2)

    def rows(c):
        return pl.ds(c * chunk, chunk)

    def copy(src, dst, sem):
        cp = pltpu.make_async_copy(src, dst, sem)
        cp.start()
        return cp

    def remote(src, dst, ssem, rsem, dev):
        return pltpu.make_async_remote_copy(
            src_ref=src, dst_ref=dst, send_sem=ssem, recv_sem=rsem,
            device_id=(dev,), device_id_type=pl.DeviceIdType.MESH,
        )

    rmax[...] = jnp.zeros_like(rmax)

    for p in range(N_PASS):
        col_cw = pl.ds(p * w, w)
        col_ccw = pl.ds(halfn + p * w, w)

        c1 = copy(p_ref.at[rows(my), col_cw], own_cw, loc_a)
        c2 = copy(p_ref.at[rows(my), col_ccw], own_ccw, loc_c)
        c1.wait()
        c2.wait()
        send_cw[...] = own_cw[...].astype(jnp.bfloat16)
        send_ccw[...] = own_ccw[...].astype(jnp.bfloat16)

        for s in range(N_DEV - 1):
            r1 = remote(send_cw, rs_cw.at[s], s_cw.at[s], r_cw.at[s], right)
            r2 = remote(send_ccw, rs_ccw.at[s], s_ccw.at[s], r_ccw.at[s], left)
            r1.start()
            r2.start()
            c_recv_cw = (my + N_DEV - s - 1) % N_DEV
            c_recv_ccw = (my + s + 1) % N_DEV
            c1 = copy(p_ref.at[rows(c_recv_cw), col_cw], own_cw, loc_a)
            c2 = copy(p_ref.at[rows(c_recv_ccw), col_ccw], own_ccw, loc_c)
            r1.wait()
            r2.wait()
            c1.wait()
            c2.wait()
            if s < N_DEV - 2:
                send_cw[...] = (
                    rs_cw[s].astype(jnp.float32) + own_cw[...]
                ).astype(jnp.bfloat16)
                send_ccw[...] = (
                    rs_ccw[s].astype(jnp.float32) + own_ccw[...]
                ).astype(jnp.bfloat16)
            else:
                own_cw[...] = rs_cw[s].astype(jnp.float32) + own_cw[...]
                own_ccw[...] = rs_ccw[s].astype(jnp.float32) + own_ccw[...]
                mx = jnp.maximum(
                    jnp.max(jnp.abs(own_cw[...])),
                    jnp.max(jnp.abs(own_ccw[...])),
                )
                rmax[...] = jnp.maximum(rmax[...], mx)
                c1 = copy(own_cw, ytmp_ref.at[0, :, pl.ds(p * w, w)], loc_b)
                c2 = copy(own_ccw, ytmp_ref.at[1, :, pl.ds(p * w, w)], loc_d)
                c1.wait()
                c2.wait()

        if p < N_PASS - 1:
            for nbr in (left, right):
                pl.semaphore_signal(
                    pass_sem, inc=1, device_id=(nbr,),
                    device_id_type=pl.DeviceIdType.MESH,
                )
            pl.semaphore_wait(pass_sem, 2)

    rowme = pl.ds(my, 1)
    amax_buf[rowme, :] = rmax[0:1, :]
    am_descs = []
    for off in (1, 2, 3):
        t = (my + off) % N_DEV
        r = remote(
            amax_buf.at[rowme], amax_buf.at[rowme],
            am_s.at[off - 1], am_r.at[my], t,
        )
        r.start()
        am_descs.append(r)
    for off in (1, 2, 3):
        pr = (my + off) % N_DEV
        d = remote(
            amax_buf.at[rowme], amax_buf.at[pl.ds(pr, 1)],
            am_s.at[0], am_r.at[pr], left,
        )
        d.wait_recv()
    for r in am_descs:
        r.wait_send()
    gmax = jnp.max(amax_buf[...])
    amax_ref[...] = jnp.zeros_like(amax_ref) + gmax

    inv = 448.0 / gmax
    f_cw = (my + 1) % N_DEV
    f_ccw = (my + N_DEV - 1) % N_DEV
    for idx, (colbase, frows) in enumerate(((0, f_cw), (halfn, f_ccw))):
        for p in range(N_PASS):
            c1 = copy(ytmp_ref.at[idx, :, pl.ds(p * w, w)], own_cw, loc_a)
            c1.wait()
            qbuf[...] = jnp.clip(own_cw[...] * inv, -448.0, 448.0).astype(
                jnp.float8_e4m3fn
            )
            c2 = copy(
                qbuf, q_ref.at[rows(frows), pl.ds(colbase + p * w, w)], loc_b
            )
            c2.wait()

    for s in range(N_DEV - 1):
        c_cw = (my + N_DEV + 1 - s) % N_DEV
        c_ccw = (my + N_DEV - 1 + s) % N_DEV
        a1 = remote(
            q_ref.at[rows(c_cw), pl.ds(0, halfn)],
            q_ref.at[rows(c_cw), pl.ds(0, halfn)],
            ag_s_cw.at[s], ag_r_cw.at[s], right,
        )
        a2 = remote(
            q_ref.at[rows(c_ccw), pl.ds(halfn, halfn)],
            q_ref.at[rows(c_ccw), pl.ds(halfn, halfn)],
            ag_s_ccw.at[s], ag_r_ccw.at[s], left,
        )
        a1.start()
        a2.start()
        a1.wait()
        a2.wait()


def _fused_ar_quant(part):
    m, n = part.shape
    chunk = m // N_DEV
    halfn = n // 2
    w = halfn // N_PASS
    return pl.pallas_call(
        _body,
        out_shape=[
            jax.ShapeDtypeStruct((m, n), jnp.float8_e4m3fn),
            jax.ShapeDtypeStruct((8, 128), jnp.float32),
            jax.ShapeDtypeStruct((2, chunk, halfn), jnp.float32),
        ],
        in_specs=[pl.BlockSpec(memory_space=pl.ANY)],
        out_specs=[
            pl.BlockSpec(memory_space=pl.ANY),
            pl.BlockSpec(memory_space=pltpu.VMEM),
            pl.BlockSpec(memory_space=pl.ANY),
        ],
        scratch_shapes=[
            pltpu.VMEM((chunk, w), jnp.bfloat16),
            pltpu.VMEM((chunk, w), jnp.float32),
            pltpu.VMEM((N_DEV - 1, chunk, w), jnp.bfloat16),
            pltpu.VMEM((chunk, w), jnp.bfloat16),
            pltpu.VMEM((chunk, w), jnp.float32),
            pltpu.VMEM((N_DEV - 1, chunk, w), jnp.bfloat16),
            pltpu.VMEM((chunk, w), jnp.float8_e4m3fn),
            pltpu.VMEM((8, 128), jnp.float32),
            pltpu.VMEM((N_DEV, 128), jnp.float32),
            pltpu.SemaphoreType.DMA((N_DEV - 1,)),
            pltpu.SemaphoreType.DMA((N_DEV - 1,)),
            pltpu.SemaphoreType.DMA((N_DEV - 1,)),
            pltpu.SemaphoreType.DMA((N_DEV - 1,)),
            pltpu.SemaphoreType.DMA((N_DEV - 1,)),
            pltpu.SemaphoreType.DMA((N_DEV,)),
            pltpu.SemaphoreType.DMA((N_DEV - 1,)),
            pltpu.SemaphoreType.DMA((N_DEV - 1,)),
            pltpu.SemaphoreType.DMA((N_DEV - 1,)),
            pltpu.SemaphoreType.DMA((N_DEV - 1,)),
            pltpu.SemaphoreType.DMA,
            pltpu.SemaphoreType.DMA,
            pltpu.SemaphoreType.DMA,
            pltpu.SemaphoreType.DMA,
            pltpu.SemaphoreType.REGULAR,
        ],
        compiler_params=pltpu.CompilerParams(collective_id=0),
    )(part)


def kernel(x, w_mat):
    part = jnp.dot(x, w_mat, preferred_element_type=jnp.float32)
    q, gmax, _ = _fused_ar_quant(part)
    scale = gmax[0, 0] / 448.0
    return q.astype(jnp.float32) * scale


# device time: 640238 ns/iter; 4.0115x vs baseline; 1.0096x over previous
import jax
import jax.numpy as jnp
from jax import lax
from jax.experimental import pallas as pl
from jax.experimental.pallas import tpu as pltpu

N_DEV = 4
N_PASS = 8


def _body(
    x_ref, w_ref, q_ref, amax_ref, ytmp_ref,
    send_cw, own_cw, rs_cw,
    send_ccw, own_ccw, rs_ccw,
    qbuf, rmax, amax_buf,
    s_cw, r_cw, s_ccw, r_ccw,
    am_s, am_r, ag_s_cw, ag_r_cw, ag_s_ccw, ag_r_ccw,
    loc_a, loc_b, loc_c, loc_d, pass_sem,
):
    m = x_ref.shape[0]
    n = w_ref.shape[1]
    chunk = m // N_DEV
    halfn = n // 2
    w = halfn // N_PASS

    my = lax.axis_index("i")
    left = (my + N_DEV - 1) % N_DEV
    right = (my + 1) % N_DEV

    barrier = pltpu.get_barrier_semaphore()
    for nbr in (left, right):
        pl.semaphore_signal(
            barrier, inc=1, device_id=(nbr,),
            device_id_type=pl.DeviceIdType.MESH,
        )
    pl.semaphore_wait(barrier, 2)

    def rows(c):
        return pl.ds(c * chunk, chunk)

    def copy(src, dst, sem):
        cp = pltpu.make_async_copy(src, dst, sem)
        cp.start()
        return cp

    def remote(src, dst, ssem, rsem, dev):
        return pltpu.make_async_remote_copy(
            src_ref=src, dst_ref=dst, send_sem=ssem, recv_sem=rsem,
            device_id=(dev,), device_id_type=pl.DeviceIdType.MESH,
        )

    rmax[...] = jnp.zeros_like(rmax)

    def ptile(c, col):
        return jnp.dot(
            x_ref[rows(c), :], w_ref[:, col],
            preferred_element_type=jnp.float32,
        )

    for p in range(N_PASS):
        col_cw = pl.ds(p * w, w)
        col_ccw = pl.ds(halfn + p * w, w)

        own_cw[...] = ptile(my, col_cw)
        own_ccw[...] = ptile(my, col_ccw)
        send_cw[...] = own_cw[...].astype(jnp.bfloat16)
        send_ccw[...] = own_ccw[...].astype(jnp.bfloat16)

        for s in range(N_DEV - 1):
            r1 = remote(send_cw, rs_cw.at[s], s_cw.at[s], r_cw.at[s], right)
            r2 = remote(send_ccw, rs_ccw.at[s], s_ccw.at[s], r_ccw.at[s], left)
            r1.start()
            r2.start()
            c_recv_cw = (my + N_DEV - s - 1) % N_DEV
            c_recv_ccw = (my + s + 1) % N_DEV
            own_cw[...] = ptile(c_recv_cw, col_cw)
            own_ccw[...] = ptile(c_recv_ccw, col_ccw)
            r1.wait()
            r2.wait()
            if s < N_DEV - 2:
                send_cw[...] = (
                    rs_cw[s].astype(jnp.float32) + own_cw[...]
                ).astype(jnp.bfloat16)
                send_ccw[...] = (
                    rs_ccw[s].astype(jnp.float32) + own_ccw[...]
                ).astype(jnp.bfloat16)
            else:
                own_cw[...] = rs_cw[s].astype(jnp.float32) + own_cw[...]
                own_ccw[...] = rs_ccw[s].astype(jnp.float32) + own_ccw[...]
                mx = jnp.maximum(
                    jnp.max(jnp.abs(own_cw[...])),
                    jnp.max(jnp.abs(own_ccw[...])),
                )
                rmax[...] = jnp.maximum(rmax[...], mx)
                c1 = copy(own_cw, ytmp_ref.at[0, :, pl.ds(p * w, w)], loc_b)
                c2 = copy(own_ccw, ytmp_ref.at[1, :, pl.ds(p * w, w)], loc_d)
                c1.wait()
                c2.wait()

        if p < N_PASS - 1:
            for nbr in (left, right):
                pl.semaphore_signal(
                    pass_sem, inc=1, device_id=(nbr,),
                    device_id_type=pl.DeviceIdType.MESH,
                )
            pl.semaphore_wait(pass_sem, 2)

    rowme = pl.ds(my, 1)
    amax_buf[rowme, :] = rmax[0:1, :]
    am_descs = []
    for off in (1, 2, 3):
        t = (my + off) % N_DEV
        r = remote(
            amax_buf.at[rowme], amax_buf.at[rowme],
            am_s.at[off - 1], am_r.at[my], t,
        )
        r.start()
        am_descs.append(r)
    for off in (1, 2, 3):
        pr = (my + off) % N_DEV
        d = remote(
            amax_buf.at[rowme], amax_buf.at[pl.ds(pr, 1)],
            am_s.at[0], am_r.at[pr], left,
        )
        d.wait_recv()
    for r in am_descs:
        r.wait_send()
    gmax = jnp.max(amax_buf[...])
    amax_ref[...] = jnp.zeros_like(amax_ref) + gmax

    inv = 448.0 / gmax
    f_cw = (my + 1) % N_DEV
    f_ccw = (my + N_DEV - 1) % N_DEV
    for idx, (colbase, frows) in enumerate(((0, f_cw), (halfn, f_ccw))):
        for p in range(N_PASS):
            c1 = copy(ytmp_ref.at[idx, :, pl.ds(p * w, w)], own_cw, loc_a)
            c1.wait()
            qbuf[...] = jnp.clip(own_cw[...] * inv, -448.0, 448.0).astype(
                jnp.float8_e4m3fn
            )
            c2 = copy(
                qbuf, q_ref.at[rows(frows), pl.ds(colbase + p * w, w)], loc_b
            )
            c2.wait()

    for s in range(N_DEV - 1):
        c_cw = (my + N_DEV + 1 - s) % N_DEV
        c_ccw = (my + N_DEV - 1 + s) % N_DEV
        a1 = remote(
            q_ref.at[rows(c_cw), pl.ds(0, halfn)],
            q_ref.at[rows(c_cw), pl.ds(0, halfn)],
            ag_s_cw.at[s], ag_r_cw.at[s], right,
        )
        a2 = remote(
            q_ref.at[rows(c_ccw), pl.ds(halfn, halfn)],
            q_ref.at[rows(c_ccw), pl.ds(halfn, halfn)],
            ag_s_ccw.at[s], ag_r_ccw.at[s], left,
        )
        a1.start()
        a2.start()
        a1.wait()
        a2.wait()


def _fused_gemm_ar_quant(xb, wb):
    m = xb.shape[0]
    n = wb.shape[1]
    chunk = m // N_DEV
    halfn = n // 2
    w = halfn // N_PASS
    return pl.pallas_call(
        _body,
        out_shape=[
            jax.ShapeDtypeStruct((m, n), jnp.float8_e4m3fn),
            jax.ShapeDtypeStruct((8, 128), jnp.float32),
            jax.ShapeDtypeStruct((2, chunk, halfn), jnp.float32),
        ],
        in_specs=[
            pl.BlockSpec(memory_space=pltpu.VMEM),
            pl.BlockSpec(memory_space=pltpu.VMEM),
        ],
        out_specs=[
            pl.BlockSpec(memory_space=pl.ANY),
            pl.BlockSpec(memory_space=pltpu.VMEM),
            pl.BlockSpec(memory_space=pl.ANY),
        ],
        scratch_shapes=[
            pltpu.VMEM((chunk, w), jnp.bfloat16),
            pltpu.VMEM((chunk, w), jnp.float32),
            pltpu.VMEM((N_DEV - 1, chunk, w), jnp.bfloat16),
            pltpu.VMEM((chunk, w), jnp.bfloat16),
            pltpu.VMEM((chunk, w), jnp.float32),
            pltpu.VMEM((N_DEV - 1, chunk, w), jnp.bfloat16),
            pltpu.VMEM((chunk, w), jnp.float8_e4m3fn),
            pltpu.VMEM((8, 128), jnp.float32),
            pltpu.VMEM((N_DEV, 128), jnp.float32),
            pltpu.SemaphoreType.DMA((N_DEV - 1,)),
            pltpu.SemaphoreType.DMA((N_DEV - 1,)),
            pltpu.SemaphoreType.DMA((N_DEV - 1,)),
            pltpu.SemaphoreType.DMA((N_DEV - 1,)),
            pltpu.SemaphoreType.DMA((N_DEV - 1,)),
            pltpu.SemaphoreType.DMA((N_DEV,)),
            pltpu.SemaphoreType.DMA((N_DEV - 1,)),
            pltpu.SemaphoreType.DMA((N_DEV - 1,)),
            pltpu.SemaphoreType.DMA((N_DEV - 1,)),
            pltpu.SemaphoreType.DMA((N_DEV - 1,)),
            pltpu.SemaphoreType.DMA,
            pltpu.SemaphoreType.DMA,
            pltpu.SemaphoreType.DMA,
            pltpu.SemaphoreType.DMA,
            pltpu.SemaphoreType.REGULAR,
        ],
        compiler_params=pltpu.CompilerParams(collective_id=0),
    )(xb, wb)


def kernel(x, w_mat):
    q, gmax, _ = _fused_gemm_ar_quant(
        x.astype(jnp.bfloat16), w_mat.astype(jnp.bfloat16)
    )
    scale = gmax[0, 0] / 448.0
    return q.astype(jnp.float32) * scale
